# baseline (speedup 1.0000x reference)
import functools

import jax
import jax.numpy as jnp
import numpy as np
from jax import lax
from jax.experimental import pallas as pl
from jax.experimental.pallas import tpu as pltpu
from jax.experimental.pallas import tpu_sc as plsc

B, N, D = 64, 64, 128
E = N * (N - 1)
NW = 32
BPW = B // NW
KC = D // 16
ROWS = 2048


def _rne16(u):
    w = lax.bitcast_convert_type(u, jnp.uint32)
    return (w + 0x7FFF + ((w >> 16) & 1)) >> 16


def _tc_pre_body(x_ref, wl_ref, wh_ref, p_ref):
    xb = x_ref[...]
    ylo = jnp.dot(xb, wl_ref[...], preferred_element_type=jnp.float32)
    yhi = jnp.dot(xb, wh_ref[...], preferred_element_type=jnp.float32)
    w = _rne16(ylo) | (_rne16(yhi) << 16)
    p_ref[...] = lax.bitcast_convert_type(w, jnp.int32)


def _tc_pre(x2, wl, wh):
    return pl.pallas_call(
        _tc_pre_body,
        grid=(B * N // ROWS,),
        in_specs=[
            pl.BlockSpec((ROWS, D), lambda i: (i, 0)),
            pl.BlockSpec((D, D), lambda i: (0, 0)),
            pl.BlockSpec((D, D), lambda i: (0, 0)),
        ],
        out_specs=pl.BlockSpec((ROWS, D), lambda i: (i, 0)),
        out_shape=jax.ShapeDtypeStruct((B * N, D), jnp.int32),
    )(x2, wl, wh)


def _sc_edge_body(p_hbm, meta_hbm, brel_hbm, out_hbm,
                  p_v0, meta_v0, p_v1, meta_v1, r_v, brel_v, sem0, sem1):
    cid = lax.axis_index("c")
    sid = lax.axis_index("s")
    wid = sid * 2 + cid

    pltpu.sync_copy(brel_hbm, brel_v)

    bufs = [(p_v0, meta_v0, sem0), (p_v1, meta_v1, sem1)]
    cps = []
    for rep in range(BPW):
        b = wid * BPW + rep
        pv, mv, sem = bufs[rep]
        cps.append([
            pltpu.async_copy(p_hbm.at[pl.ds(b * N, N)], pv, sem),
            pltpu.async_copy(meta_hbm.at[pl.ds(b * E, E)], mv.at[pl.ds(0, E)], sem)])

    for rep in range(BPW):
        b = wid * BPW + rep
        p_v, meta_v, _ = bufs[rep]
        for cp in cps[rep]:
            cp.wait()

        def n_body(n, carry):
            zero_bf = plsc.bitcast(jnp.zeros((16,), jnp.int32), jnp.bfloat16)
            brel = [plsc.bitcast(brel_v[pl.ds(16 * c, 16)], jnp.bfloat16)
                    for c in range(4)]
            e0 = n * (N - 1)
            mc = [meta_v[pl.ds(e0 + 16 * j, 16)] for j in range(4)]
            acc = [jnp.zeros((16,), jnp.float32) for _ in range(KC)]
            for g0 in range(0, N - 1, 16):
                accb = [zero_bf for _ in range(4)]
                for m in range(g0, min(g0 + 16, N - 1)):
                    j, l = divmod(m, 16)
                    w = mc[j][l]
                    i1 = w & (N - 1)
                    i2 = (w >> 6) & (N - 1)
                    sm = w >> 16
                    sw = sm | (sm << 16)
                    sb = plsc.bitcast(jnp.full((16,), sw, jnp.int32),
                                      jnp.bfloat16)
                    for c in range(4):
                        a = plsc.bitcast(p_v[i1, pl.ds(16 * c, 16)],
                                         jnp.bfloat16)
                        b2 = plsc.bitcast(p_v[i2, pl.ds(D // 2 + 16 * c, 16)],
                                          jnp.bfloat16)
                        t = jnp.maximum((a + b2) * sb + brel[c], zero_bf)
                        accb[c] = accb[c] + t
                for c in range(4):
                    lo, hi = plsc.unpack(accb[c],
                                         format=plsc.PackFormat.INTERLEAVED,
                                         preferred_element_type=jnp.float32)
                    acc[c] = acc[c] + lo
                    acc[c + 4] = acc[c + 4] + hi
            for k in range(KC):
                r_v[n, pl.ds(16 * k, 16)] = acc[k]
            return carry

        lax.fori_loop(0, N, n_body, 0)
        pltpu.sync_copy(r_v, out_hbm.at[pl.ds(b * N, N)])


def _sc_edge(p, meta, brelw):
    mesh = plsc.VectorSubcoreMesh(core_axis_name="c", subcore_axis_name="s")
    f = pl.kernel(
        _sc_edge_body,
        out_type=jax.ShapeDtypeStruct((B * N, D), jnp.float32),
        mesh=mesh,
        compiler_params=pltpu.CompilerParams(needs_layout_passes=False),
        scratch_types=[
            pltpu.VMEM((N, D), jnp.int32),
            pltpu.VMEM((E + 16,), jnp.int32),
            pltpu.VMEM((N, D), jnp.int32),
            pltpu.VMEM((E + 16,), jnp.int32),
            pltpu.VMEM((N, D), jnp.float32),
            pltpu.VMEM((D // 2,), jnp.int32),
            pltpu.SemaphoreType.DMA,
            pltpu.SemaphoreType.DMA,
        ],
    )
    return f(p, meta, brelw)


def _tc_post_body(x_ref, r_ref, ws_ref, bs_ref, wa_ref, ba_ref,
                  wg1_ref, wg2_ref, bg_ref, o_ref):
    xb = x_ref[...]
    xs = jnp.maximum(
        jnp.dot(xb, ws_ref[...], preferred_element_type=jnp.float32)
        + bs_ref[...], 0.0)
    pred = xs + r_ref[...]
    a = jnp.maximum(
        jnp.dot(pred, wa_ref[...], preferred_element_type=jnp.float32)
        + ba_ref[...], 0.0)
    o = (jnp.dot(a, wg1_ref[...], preferred_element_type=jnp.float32)
         + jnp.dot(xb, wg2_ref[...], preferred_element_type=jnp.float32)
         + bg_ref[...])
    o_ref[...] = jnp.maximum(o, 0.0)


def _tc_post(x2, r2, ws, bs, wa, ba, wg1, wg2, bg):
    full = lambda shape: pl.BlockSpec(shape, lambda i: tuple(0 for _ in shape))
    return pl.pallas_call(
        _tc_post_body,
        grid=(B * N // ROWS,),
        in_specs=[
            pl.BlockSpec((ROWS, D), lambda i: (i, 0)),
            pl.BlockSpec((ROWS, D), lambda i: (i, 0)),
            full((D, D)), full((1, D)),
            full((D, D)), full((1, D)),
            full((D, D)), full((D, D)), full((1, D)),
        ],
        out_specs=pl.BlockSpec((ROWS, D), lambda i: (i, 0)),
        out_shape=jax.ShapeDtypeStruct((B * N, D), jnp.float32),
    )(x2, r2, ws, bs, wa, ba, wg1, wg2, bg)


def kernel(x, g_idx, W_self, b_self, W_rel, b_rel, W_aff, b_aff,
           W_agg, b_agg):
    g = g_idx.astype(jnp.int32).reshape(B * E, 3)
    sbits = lax.bitcast_convert_type(
        g[:, 2].astype(jnp.float32).astype(jnp.bfloat16),
        jnp.uint16).astype(jnp.int32)
    meta = ((g[:, 0] & (N - 1)) | ((g[:, 1] & (N - 1)) << 6)
            | (sbits << 16))

    H = D // 2
    wl = jnp.concatenate([W_rel[:D, :H], W_rel[D:, :H]], axis=1)
    wh = jnp.concatenate([W_rel[:D, H:], W_rel[D:, H:]], axis=1)
    b16 = b_rel.astype(jnp.bfloat16)
    brelw = lax.bitcast_convert_type(
        jnp.stack([b16[:H], b16[H:]], axis=-1), jnp.int32)

    x2 = x.reshape(B * N, D)
    p = _tc_pre(x2, wl, wh)
    r2 = _sc_edge(p, meta, brelw)
    out2 = _tc_post(x2, r2,
                    W_self, b_self.reshape(1, D),
                    W_aff, b_aff.reshape(1, D),
                    W_agg[:D], W_agg[D:], b_agg.reshape(1, D))
    return out2.reshape(B, N, D)


# speedup vs baseline: 1.0631x; 1.0631x over previous
import functools

import jax
import jax.numpy as jnp
import numpy as np
from jax import lax
from jax.experimental import pallas as pl
from jax.experimental.pallas import tpu as pltpu
from jax.experimental.pallas import tpu_sc as plsc

B, N, D = 64, 64, 128
E = N * (N - 1)
NW = 32
BPW = B // NW
KC = D // 16
ROWS = 2048


def _rne16(u):
    w = lax.bitcast_convert_type(u, jnp.uint32)
    return (w + 0x7FFF + ((w >> 16) & 1)) >> 16


def _tc_pre_body(x_ref, wl_ref, wh_ref, p_ref):
    xb = x_ref[...]
    ylo = jnp.dot(xb, wl_ref[...], preferred_element_type=jnp.float32)
    yhi = jnp.dot(xb, wh_ref[...], preferred_element_type=jnp.float32)
    w = _rne16(ylo) | (_rne16(yhi) << 16)
    p_ref[...] = lax.bitcast_convert_type(w, jnp.int32)


def _tc_pre(x2, wl, wh):
    return pl.pallas_call(
        _tc_pre_body,
        grid=(B * N // ROWS,),
        in_specs=[
            pl.BlockSpec((ROWS, D), lambda i: (i, 0)),
            pl.BlockSpec((D, D), lambda i: (0, 0)),
            pl.BlockSpec((D, D), lambda i: (0, 0)),
        ],
        out_specs=pl.BlockSpec((ROWS, D), lambda i: (i, 0)),
        out_shape=jax.ShapeDtypeStruct((B * N, D), jnp.int32),
    )(x2, wl, wh)


def _sc_edge_body(p_hbm, meta_hbm, brel_hbm, out_hbm,
                  p_v0, meta_v0, p_v1, meta_v1, r_v, brel_v, sem0, sem1):
    cid = lax.axis_index("c")
    sid = lax.axis_index("s")
    wid = sid * 2 + cid

    pltpu.sync_copy(brel_hbm, brel_v)

    bufs = [(p_v0, meta_v0, sem0), (p_v1, meta_v1, sem1)]
    cps = []
    for rep in range(BPW):
        b = wid * BPW + rep
        pv, mv, sem = bufs[rep]
        cps.append([
            pltpu.async_copy(p_hbm.at[pl.ds(b * N, N)], pv, sem),
            pltpu.async_copy(meta_hbm.at[pl.ds(b * E, E)], mv.at[pl.ds(0, E)], sem)])

    for rep in range(BPW):
        b = wid * BPW + rep
        p_v, meta_v, _ = bufs[rep]
        for cp in cps[rep]:
            cp.wait()

        def n_body(n, carry):
            zero_bf = plsc.bitcast(jnp.zeros((16,), jnp.int32), jnp.bfloat16)
            brel = [plsc.bitcast(brel_v[pl.ds(16 * c, 16)], jnp.bfloat16)
                    for c in range(4)]
            e0 = n * (N - 1)
            mc = [meta_v[pl.ds(e0 + 16 * j, 16)] for j in range(4)]
            acc = [jnp.zeros((16,), jnp.float32) for _ in range(KC)]
            for g0 in range(0, N - 1, 16):
                accb = [zero_bf for _ in range(4)]
                for m in range(g0, min(g0 + 16, N - 1)):
                    j, l = divmod(m, 16)
                    w = mc[j][l]
                    i1 = w & (N - 1)
                    i2 = (w >> 6) & (N - 1)
                    sm = w >> 16
                    sw = sm | (sm << 16)
                    sb = plsc.bitcast(jnp.full((16,), sw, jnp.int32),
                                      jnp.bfloat16)
                    for c in range(4):
                        a = plsc.bitcast(p_v[i1, pl.ds(16 * c, 16)],
                                         jnp.bfloat16)
                        b2 = plsc.bitcast(p_v[i2, pl.ds(D // 2 + 16 * c, 16)],
                                          jnp.bfloat16)
                        t = jnp.maximum((a + b2) * sb + brel[c], zero_bf)
                        accb[c] = accb[c] + t
                for c in range(4):
                    lo, hi = plsc.unpack(accb[c],
                                         format=plsc.PackFormat.INTERLEAVED,
                                         preferred_element_type=jnp.float32)
                    acc[c] = acc[c] + lo
                    acc[c + 4] = acc[c + 4] + hi
            for k in range(KC):
                r_v[n, pl.ds(16 * k, 16)] = acc[k]
            return carry

        lax.fori_loop(0, N, n_body, 0)
        pltpu.sync_copy(r_v, out_hbm.at[pl.ds(b * N, N)])


def _sc_edge(p, meta, brelw):
    mesh = plsc.VectorSubcoreMesh(core_axis_name="c", subcore_axis_name="s")
    f = pl.kernel(
        _sc_edge_body,
        out_type=jax.ShapeDtypeStruct((B * N, D), jnp.float32),
        mesh=mesh,
        compiler_params=pltpu.CompilerParams(needs_layout_passes=False),
        scratch_types=[
            pltpu.VMEM((N, D), jnp.int32),
            pltpu.VMEM((E + 16,), jnp.int32),
            pltpu.VMEM((N, D), jnp.int32),
            pltpu.VMEM((E + 16,), jnp.int32),
            pltpu.VMEM((N, D), jnp.float32),
            pltpu.VMEM((D // 2,), jnp.int32),
            pltpu.SemaphoreType.DMA,
            pltpu.SemaphoreType.DMA,
        ],
    )
    return f(p, meta, brelw)


def _tc_post_body(x_ref, r_ref, ws_ref, bs_ref, wa_ref, ba_ref,
                  wg1_ref, wg2_ref, bg_ref, o_ref):
    xb = x_ref[...]
    xs = jnp.maximum(
        jnp.dot(xb, ws_ref[...], preferred_element_type=jnp.float32)
        + bs_ref[...], 0.0)
    pred = xs + r_ref[...]
    a = jnp.maximum(
        jnp.dot(pred, wa_ref[...], preferred_element_type=jnp.float32)
        + ba_ref[...], 0.0)
    o = (jnp.dot(a, wg1_ref[...], preferred_element_type=jnp.float32)
         + jnp.dot(xb, wg2_ref[...], preferred_element_type=jnp.float32)
         + bg_ref[...])
    o_ref[...] = jnp.maximum(o, 0.0)


def _tc_post(x2, r2, ws, bs, wa, ba, wg1, wg2, bg):
    full = lambda shape: pl.BlockSpec(shape, lambda i: tuple(0 for _ in shape))
    return pl.pallas_call(
        _tc_post_body,
        grid=(B * N // ROWS,),
        in_specs=[
            pl.BlockSpec((ROWS, D), lambda i: (i, 0)),
            pl.BlockSpec((ROWS, D), lambda i: (i, 0)),
            full((D, D)), full((1, D)),
            full((D, D)), full((1, D)),
            full((D, D)), full((D, D)), full((1, D)),
        ],
        out_specs=pl.BlockSpec((ROWS, D), lambda i: (i, 0)),
        out_shape=jax.ShapeDtypeStruct((B * N, D), jnp.float32),
    )(x2, r2, ws, bs, wa, ba, wg1, wg2, bg)


def kernel(x, g_idx, W_self, b_self, W_rel, b_rel, W_aff, b_aff,
           W_agg, b_agg):
    g = g_idx.astype(jnp.int32)
    sbits = lax.bitcast_convert_type(
        g[..., 2].astype(jnp.float32).astype(jnp.bfloat16),
        jnp.uint16).astype(jnp.int32)
    w = (g[..., 0] & (N - 1)) | ((g[..., 1] & (N - 1)) << 6) | (sbits << 16)
    meta = w.reshape(B * E)

    H = D // 2
    wl = jnp.concatenate([W_rel[:D, :H], W_rel[D:, :H]], axis=1)
    wh = jnp.concatenate([W_rel[:D, H:], W_rel[D:, H:]], axis=1)
    b16 = b_rel.astype(jnp.bfloat16)
    brelw = lax.bitcast_convert_type(
        jnp.stack([b16[:H], b16[H:]], axis=-1), jnp.int32)

    x2 = x.reshape(B * N, D)
    p = _tc_pre(x2, wl, wh)
    r2 = _sc_edge(p, meta, brelw)
    out2 = _tc_post(x2, r2,
                    W_self, b_self.reshape(1, D),
                    W_aff, b_aff.reshape(1, D),
                    W_agg[:D], W_agg[D:], b_agg.reshape(1, D))
    return out2.reshape(B, N, D)


# speedup vs baseline: 1.0644x; 1.0012x over previous
import jax
import jax.numpy as jnp
from jax import lax
from jax.experimental import pallas as pl
from jax.experimental.pallas import tpu as pltpu
from jax.experimental.pallas import tpu_sc as plsc

B, N, D = 64, 64, 128
E = N * (N - 1)
NW = 32
BPW = B // NW
KC = D // 16
ROWS = 2048


def _rne16(u):
    w = lax.bitcast_convert_type(u, jnp.uint32)
    return (w + 0x7FFF + ((w >> 16) & 1)) >> 16


def _tc_pre_body(x_ref, wl_ref, wh_ref, p_ref):
    xb = x_ref[...]
    ylo = jnp.dot(xb, wl_ref[...], preferred_element_type=jnp.float32)
    yhi = jnp.dot(xb, wh_ref[...], preferred_element_type=jnp.float32)
    w = _rne16(ylo) | (_rne16(yhi) << 16)
    p_ref[...] = lax.bitcast_convert_type(w, jnp.int32)


def _tc_pre(x2, wl, wh):
    return pl.pallas_call(
        _tc_pre_body,
        grid=(B * N // ROWS,),
        in_specs=[
            pl.BlockSpec((ROWS, D), lambda i: (i, 0)),
            pl.BlockSpec((D, D), lambda i: (0, 0)),
            pl.BlockSpec((D, D), lambda i: (0, 0)),
        ],
        out_specs=pl.BlockSpec((ROWS, D), lambda i: (i, 0)),
        out_shape=jax.ShapeDtypeStruct((B * N, D), jnp.int32),
    )(x2, wl, wh)


def _sc_edge_body(p_hbm, meta_hbm, brel_hbm, out_hbm,
                  p_v0, meta_v0, p_v1, meta_v1, r_v, brel_v, sem0, sem1):
    cid = lax.axis_index("c")
    sid = lax.axis_index("s")
    wid = sid * 2 + cid

    pltpu.sync_copy(brel_hbm, brel_v)

    bufs = [(p_v0, meta_v0, sem0), (p_v1, meta_v1, sem1)]
    cps = []
    for rep in range(BPW):
        b = wid * BPW + rep
        pv, mv, sem = bufs[rep]
        cps.append([
            pltpu.async_copy(p_hbm.at[pl.ds(b * N, N)], pv, sem),
            pltpu.async_copy(meta_hbm.at[pl.ds(b * E, E)], mv.at[pl.ds(0, E)], sem)])

    for rep in range(BPW):
        b = wid * BPW + rep
        p_v, meta_v, _ = bufs[rep]
        for cp in cps[rep]:
            cp.wait()

        def n_body(n, carry):
            zero_bf = plsc.bitcast(jnp.zeros((16,), jnp.int32), jnp.bfloat16)
            brel = [plsc.bitcast(brel_v[pl.ds(16 * c, 16)], jnp.bfloat16)
                    for c in range(4)]
            e0 = n * (N - 1)
            mc = [meta_v[pl.ds(e0 + 16 * j, 16)] for j in range(4)]
            acc = [jnp.zeros((16,), jnp.float32) for _ in range(KC)]
            for g0 in range(0, N - 1, 16):
                accb = [zero_bf for _ in range(4)]
                for m in range(g0, min(g0 + 16, N - 1)):
                    j, l = divmod(m, 16)
                    w = mc[j][l]
                    i1 = w & (N - 1)
                    i2 = (w >> 6) & (N - 1)
                    sm = w >> 16
                    sw = sm | (sm << 16)
                    sb = plsc.bitcast(jnp.full((16,), sw, jnp.int32),
                                      jnp.bfloat16)
                    for c in range(4):
                        a = plsc.bitcast(p_v[i1, pl.ds(16 * c, 16)],
                                         jnp.bfloat16)
                        b2 = plsc.bitcast(p_v[i2, pl.ds(D // 2 + 16 * c, 16)],
                                          jnp.bfloat16)
                        t = jnp.maximum((a + b2) * sb + brel[c], zero_bf)
                        accb[c] = accb[c] + t
                for c in range(4):
                    lo, hi = plsc.unpack(accb[c],
                                         format=plsc.PackFormat.INTERLEAVED,
                                         preferred_element_type=jnp.float32)
                    acc[c] = acc[c] + lo
                    acc[c + 4] = acc[c + 4] + hi
            for k in range(KC):
                r_v[n, pl.ds(16 * k, 16)] = acc[k]
            return carry

        lax.fori_loop(0, N, n_body, 0)
        pltpu.sync_copy(r_v, out_hbm.at[pl.ds(b * N, N)])


def _sc_edge(p, meta, brelw):
    mesh = plsc.VectorSubcoreMesh(core_axis_name="c", subcore_axis_name="s")
    f = pl.kernel(
        _sc_edge_body,
        out_type=jax.ShapeDtypeStruct((B * N, D), jnp.float32),
        mesh=mesh,
        compiler_params=pltpu.CompilerParams(needs_layout_passes=False),
        scratch_types=[
            pltpu.VMEM((N, D), jnp.int32),
            pltpu.VMEM((E + 16,), jnp.int32),
            pltpu.VMEM((N, D), jnp.int32),
            pltpu.VMEM((E + 16,), jnp.int32),
            pltpu.VMEM((N, D), jnp.float32),
            pltpu.VMEM((D // 2,), jnp.int32),
            pltpu.SemaphoreType.DMA,
            pltpu.SemaphoreType.DMA,
        ],
    )
    return f(p, meta, brelw)


def _tc_post_body(x_ref, r_ref, ws_ref, bs_ref, wa_ref, ba_ref,
                  wg1_ref, wg2_ref, bg_ref, o_ref):
    xb = x_ref[...]
    xs = jnp.maximum(
        jnp.dot(xb, ws_ref[...], preferred_element_type=jnp.float32)
        + bs_ref[...], 0.0)
    pred = xs + r_ref[...]
    a = jnp.maximum(
        jnp.dot(pred, wa_ref[...], preferred_element_type=jnp.float32)
        + ba_ref[...], 0.0)
    o = (jnp.dot(a, wg1_ref[...], preferred_element_type=jnp.float32)
         + jnp.dot(xb, wg2_ref[...], preferred_element_type=jnp.float32)
         + bg_ref[...])
    o_ref[...] = jnp.maximum(o, 0.0)


def _tc_post(x2, r2, ws, bs, wa, ba, wg1, wg2, bg):
    full = lambda shape: pl.BlockSpec(shape, lambda i: tuple(0 for _ in shape))
    return pl.pallas_call(
        _tc_post_body,
        grid=(B * N // ROWS,),
        in_specs=[
            pl.BlockSpec((ROWS, D), lambda i: (i, 0)),
            pl.BlockSpec((ROWS, D), lambda i: (i, 0)),
            full((D, D)), full((1, D)),
            full((D, D)), full((1, D)),
            full((D, D)), full((D, D)), full((1, D)),
        ],
        out_specs=pl.BlockSpec((ROWS, D), lambda i: (i, 0)),
        out_shape=jax.ShapeDtypeStruct((B * N, D), jnp.float32),
    )(x2, r2, ws, bs, wa, ba, wg1, wg2, bg)


def kernel(x, g_idx, W_self, b_self, W_rel, b_rel, W_aff, b_aff,
           W_agg, b_agg):
    g = g_idx.astype(jnp.int32)
    sbits = lax.bitcast_convert_type(
        g[..., 2].astype(jnp.float32).astype(jnp.bfloat16),
        jnp.uint16).astype(jnp.int32)
    w = (g[..., 0] & (N - 1)) | ((g[..., 1] & (N - 1)) << 6) | (sbits << 16)
    meta = w.reshape(B * E)

    H = D // 2
    wl = jnp.concatenate([W_rel[:D, :H], W_rel[D:, :H]], axis=1)
    wh = jnp.concatenate([W_rel[:D, H:], W_rel[D:, H:]], axis=1)
    b16 = b_rel.astype(jnp.bfloat16)
    brelw = lax.bitcast_convert_type(
        jnp.stack([b16[:H], b16[H:]], axis=-1), jnp.int32)

    x2 = x.reshape(B * N, D)
    p = _tc_pre(x2, wl, wh)
    r2 = _sc_edge(p, meta, brelw)
    out2 = _tc_post(x2, r2,
                    W_self, b_self.reshape(1, D),
                    W_aff, b_aff.reshape(1, D),
                    W_agg[:D], W_agg[D:], b_agg.reshape(1, D))
    return out2.reshape(B, N, D)
